# baseline (speedup 1.0000x reference)
import functools

import jax
import jax.numpy as jnp
from jax import lax
from jax.experimental import pallas as pl
from jax.experimental.pallas import tpu as pltpu
from jax.experimental.pallas import tpu_sc as plsc

VOCAB = 1000000
D = 64
B = 1024
S = 200
NC = 2
NS = 16
NW = NC * NS
SEQ_PER_W = B // NW
ROWS_PER_W = SEQ_PER_W * S
NCH = SEQ_PER_W
SCALE = 8.0


def _body(idx_hbm, emb_hbm, out_hbm, idx_v, g_buf, gs):
    wid = lax.axis_index("s") * NC + lax.axis_index("c")
    base_w = wid * ROWS_PER_W

    for t in range(NCH):
        pltpu.sync_copy(idx_hbm.at[pl.ds(base_w + t * S, S)], idx_v)
        pltpu.async_copy(emb_hbm.at[idx_v], g_buf, gs).wait()
        pltpu.sync_copy(g_buf, out_hbm.at[pl.ds(base_w + t * S, S)])


@jax.jit
def _embed(idx, emb):
    mesh = plsc.VectorSubcoreMesh(core_axis_name="c", subcore_axis_name="s")
    f = pl.kernel(
        _body,
        out_type=jax.ShapeDtypeStruct((B * S, D), jnp.float32),
        mesh=mesh,
        scratch_types=[
            pltpu.VMEM((S,), jnp.int32),
            pltpu.VMEM((S, D), jnp.float32),
            pltpu.SemaphoreType.DMA,
        ],
        compiler_params=pltpu.CompilerParams(use_tc_tiling_on_sc=False),
    )
    return f(idx, emb)


def _fma_body(x_ref, pe_ref, o_ref):
    o_ref[...] = x_ref[...] * SCALE + pe_ref[...]


ROWS_PER_BLK = 8


@jax.jit
def _fma(x, pe):
    return pl.pallas_call(
        _fma_body,
        out_shape=jax.ShapeDtypeStruct((B, S, D), jnp.float32),
        grid=(B // ROWS_PER_BLK,),
        in_specs=[
            pl.BlockSpec((ROWS_PER_BLK, S, D), lambda i: (i, 0, 0)),
            pl.BlockSpec((1, S, D), lambda i: (0, 0, 0)),
        ],
        out_specs=pl.BlockSpec((ROWS_PER_BLK, S, D), lambda i: (i, 0, 0)),
    )(x, pe)


def kernel(token_sequences, embedding, positional_embedding):
    idx = token_sequences.reshape(-1).astype(jnp.int32)
    pe = positional_embedding[:, :S, :]
    out = _embed(idx, embedding)
    return _fma(out.reshape(B, S, D), pe)
